# Initial kernel scaffold; baseline (speedup 1.0000x reference)
import functools

import jax
import jax.numpy as jnp
from jax import lax
from jax.experimental import pallas as pl
from jax.experimental.pallas import tpu as pltpu
from jax.experimental.pallas import tpu_sc as plsc

VOCAB = 1000000
DIM = 64
B = 16384
K_NEG = 20

NUM_WORKERS = 32
PER_W = B // NUM_WORKERS
CHUNK = 64
N_CHUNKS = PER_W // CHUNK
NEG_CHUNK = CHUNK * K_NEG
L = 16
DC = DIM // L
IDX_SLICE = 128


def _rowsum(tile_ref):
    rows = lax.iota(jnp.int32, L)
    acc = None
    for c in range(L):
        col = jnp.full((L,), c, jnp.int32)
        g = plsc.load_gather(tile_ref, [rows, col])
        acc = g if acc is None else acc + g
    return acc


def _sc_kernel(cen_idx, ctx_idx, neg_idx, w_in, w_out, pos_out, neg_out,
               cen_idx_v, ctx_idx_v, neg_idx_v,
               cen_v, ctx_v, neg_v, pos_st, neg_st, tile, sem):
    wid = lax.axis_index("s") * 2 + lax.axis_index("c")

    def chunk_body(ci, carry):
        base = wid * PER_W + ci * CHUNK

        pltpu.sync_copy(cen_idx.at[pl.ds(base, CHUNK)], cen_idx_v)
        pltpu.sync_copy(ctx_idx.at[pl.ds(base, CHUNK)], ctx_idx_v)
        pltpu.sync_copy(neg_idx.at[pl.ds(base * K_NEG, NEG_CHUNK)], neg_idx_v)

        cps = [pltpu.async_copy(w_in.at[cen_idx_v], cen_v, sem),
               pltpu.async_copy(w_out.at[ctx_idx_v], ctx_v, sem)]
        for t in range(NEG_CHUNK // IDX_SLICE):
            cps.append(pltpu.async_copy(
                w_out.at[neg_idx_v.at[pl.ds(t * IDX_SLICE, IDX_SLICE)]],
                neg_v.at[pl.ds(t * IDX_SLICE, IDX_SLICE)], sem))
        for cp in cps:
            cp.wait()

        def pos_group(i, c2):
            for j in range(L):
                e = i * L + j
                acc = None
                for c in range(DC):
                    a = cen_v[e, pl.ds(L * c, L)]
                    b = ctx_v[e, pl.ds(L * c, L)]
                    acc = a * b if acc is None else acc + a * b
                tile[j, :] = acc
            pos_st[pl.ds(i * L, L)] = _rowsum(tile)
            return c2

        lax.fori_loop(0, CHUNK // L, pos_group, 0)

        def neg_group(g, c2):
            for eo in range(4):
                e = g * 4 + eo
                cen_c = [cen_v[e, pl.ds(L * c, L)] for c in range(DC)]
                for k in range(K_NEG):
                    sl = eo * K_NEG + k
                    srow = g * 80 + sl
                    acc = None
                    for c in range(DC):
                        n = neg_v[srow, pl.ds(L * c, L)]
                        acc = (n * cen_c[c] if acc is None
                               else acc + n * cen_c[c])
                    tile[sl % L, :] = acc
                    if sl % L == L - 1:
                        neg_st[pl.ds(g * 80 + (sl // L) * L, L)] = _rowsum(tile)
            return c2

        lax.fori_loop(0, NEG_CHUNK // 80, neg_group, 0)

        pltpu.sync_copy(pos_st, pos_out.at[pl.ds(base, CHUNK)])
        pltpu.sync_copy(neg_st, neg_out.at[pl.ds(base * K_NEG, NEG_CHUNK)])
        return carry

    lax.fori_loop(0, N_CHUNKS, chunk_body, 0)


_sc_call = functools.partial(
    pl.kernel,
    mesh=plsc.VectorSubcoreMesh(core_axis_name="c", subcore_axis_name="s"),
    out_type=[jax.ShapeDtypeStruct((B,), jnp.float32),
              jax.ShapeDtypeStruct((B * K_NEG,), jnp.float32)],
    scratch_types=[
        pltpu.VMEM((CHUNK,), jnp.int32),
        pltpu.VMEM((CHUNK,), jnp.int32),
        pltpu.VMEM((NEG_CHUNK,), jnp.int32),
        pltpu.VMEM((CHUNK, DIM), jnp.float32),
        pltpu.VMEM((CHUNK, DIM), jnp.float32),
        pltpu.VMEM((NEG_CHUNK, DIM), jnp.float32),
        pltpu.VMEM((CHUNK,), jnp.float32),
        pltpu.VMEM((NEG_CHUNK,), jnp.float32),
        pltpu.VMEM((L, L), jnp.float32),
        pltpu.SemaphoreType.DMA,
    ],
)(_sc_kernel)


@jax.jit
def kernel(center_words, context_words, negative_samples, W_in, W_out):
    cen = center_words.reshape(B).astype(jnp.int32)
    ctx = context_words.reshape(B).astype(jnp.int32)
    neg = negative_samples.reshape(B * K_NEG).astype(jnp.int32)
    pos, negf = _sc_call(cen, ctx, neg, W_in, W_out)
    return pos, negf.reshape(B, K_NEG)


# speedup vs baseline: 4.8671x; 4.8671x over previous
import functools

import jax
import jax.numpy as jnp
from jax import lax
from jax.experimental import pallas as pl
from jax.experimental.pallas import tpu as pltpu
from jax.experimental.pallas import tpu_sc as plsc

VOCAB = 1000000
DIM = 64
B = 16384
K_NEG = 20

NUM_WORKERS = 32
PER_W = B // NUM_WORKERS
CHUNK = 64
N_CHUNKS = PER_W // CHUNK
NEG_CHUNK = CHUNK * K_NEG
L = 16
DC = DIM // L
IDX_SLICE = 128


def _rowsum(tile_ref):
    rows = lax.iota(jnp.int32, L)
    acc = None
    for c in range(L):
        col = jnp.full((L,), c, jnp.int32)
        g = plsc.load_gather(tile_ref, [rows, col])
        acc = g if acc is None else acc + g
    return acc


def _sc_kernel(cen_idx, ctx_idx, neg_idx, w_in, w_out, pos_out, neg_out,
               cen_idx_v, ctx_idx_v, neg_idx_v,
               cen_v, ctx_v, neg_v, pos_st, neg_st, tile, sem):
    wid = lax.axis_index("s") * 2 + lax.axis_index("c")

    def chunk_body(ci, carry):
        base = wid * PER_W + ci * CHUNK

        pltpu.sync_copy(cen_idx.at[pl.ds(base, CHUNK)], cen_idx_v)
        pltpu.sync_copy(ctx_idx.at[pl.ds(base, CHUNK)], ctx_idx_v)
        pltpu.sync_copy(neg_idx.at[pl.ds(base * K_NEG, NEG_CHUNK)], neg_idx_v)

        cps = [pltpu.async_copy(w_in.at[cen_idx_v], cen_v, sem),
               pltpu.async_copy(w_out.at[ctx_idx_v], ctx_v, sem)]
        for t in range(NEG_CHUNK // IDX_SLICE):
            cps.append(pltpu.async_copy(
                w_out.at[neg_idx_v.at[pl.ds(t * IDX_SLICE, IDX_SLICE)]],
                neg_v.at[pl.ds(t * IDX_SLICE, IDX_SLICE)], sem))
        for cp in cps:
            cp.wait()

        def pos_group(i, c2):
            for j in range(L):
                e = i * L + j
                acc = None
                for c in range(DC):
                    a = cen_v[e, pl.ds(L * c, L)]
                    b = ctx_v[e, pl.ds(L * c, L)]
                    acc = a * b if acc is None else acc + a * b
                tile[j, :] = acc
            pos_st[pl.ds(i * L, L)] = _rowsum(tile)
            return c2

        lax.fori_loop(0, CHUNK // L, pos_group, 0)

        def neg_group(g, c2):
            for eo in range(4):
                e = g * 4 + eo
                cen_c = [cen_v[e, pl.ds(L * c, L)] for c in range(DC)]
                for k in range(K_NEG):
                    sl = eo * K_NEG + k
                    srow = g * 80 + sl
                    acc = None
                    for c in range(DC):
                        n = neg_v[srow, pl.ds(L * c, L)]
                        acc = (n * cen_c[c] if acc is None
                               else acc + n * cen_c[c])
                    tile[sl % L, :] = acc
                    if sl % L == L - 1:
                        neg_st[pl.ds(g * 80 + (sl // L) * L, L)] = _rowsum(tile)
            return c2

        lax.fori_loop(0, NEG_CHUNK // 80, neg_group, 0)

        pltpu.sync_copy(pos_st, pos_out.at[pl.ds(base, CHUNK)])
        pltpu.sync_copy(neg_st, neg_out.at[pl.ds(base * K_NEG, NEG_CHUNK)])
        return carry

    lax.fori_loop(0, N_CHUNKS, chunk_body, 0)


_sc_call = functools.partial(
    pl.kernel,
    mesh=plsc.VectorSubcoreMesh(core_axis_name="c", subcore_axis_name="s"),
    compiler_params=pltpu.CompilerParams(needs_layout_passes=False,
                                         use_tc_tiling_on_sc=False),
    out_type=[jax.ShapeDtypeStruct((B,), jnp.float32),
              jax.ShapeDtypeStruct((B * K_NEG,), jnp.float32)],
    scratch_types=[
        pltpu.VMEM((CHUNK,), jnp.int32),
        pltpu.VMEM((CHUNK,), jnp.int32),
        pltpu.VMEM((NEG_CHUNK,), jnp.int32),
        pltpu.VMEM((CHUNK, DIM), jnp.float32),
        pltpu.VMEM((CHUNK, DIM), jnp.float32),
        pltpu.VMEM((NEG_CHUNK, DIM), jnp.float32),
        pltpu.VMEM((CHUNK,), jnp.float32),
        pltpu.VMEM((NEG_CHUNK,), jnp.float32),
        pltpu.VMEM((L, L), jnp.float32),
        pltpu.SemaphoreType.DMA,
    ],
)(_sc_kernel)


@jax.jit
def kernel(center_words, context_words, negative_samples, W_in, W_out):
    cen = center_words.reshape(B).astype(jnp.int32)
    ctx = context_words.reshape(B).astype(jnp.int32)
    neg = negative_samples.reshape(B * K_NEG).astype(jnp.int32)
    pos, negf = _sc_call(cen, ctx, neg, W_in, W_out)
    return pos, negf.reshape(B, K_NEG)


# speedup vs baseline: 9.4276x; 1.9370x over previous
import functools

import jax
import jax.numpy as jnp
from jax import lax
from jax.experimental import pallas as pl
from jax.experimental.pallas import tpu as pltpu
from jax.experimental.pallas import tpu_sc as plsc

VOCAB = 1000000
DIM = 64
B = 16384
K_NEG = 20

NUM_WORKERS = 32
PER_W = B // NUM_WORKERS
CHUNK = 32
N_CHUNKS = PER_W // CHUNK
NEG_CHUNK = CHUNK * K_NEG
L = 16
DC = DIM // L
IDX_SLICE = 128
PAIR = 2 * DIM

BLK = 128
FULL_BLOCKS = VOCAB // BLK
TAIL_COLS = VOCAB - FULL_BLOCKS * BLK
K_MAIN = FULL_BLOCKS // NUM_WORKERS
EXTRA = FULL_BLOCKS - K_MAIN * NUM_WORKERS


PITCH = BLK + 1


def _transpose_block(slab, pad1, rows_out):
    iota = lax.iota(jnp.int32, L)
    seg = [iota + L * c for c in range(BLK // L)]
    colbase = [(iota + L * c) * PITCH for c in range(DC)]

    @plsc.parallel_loop(0, DIM, unroll=4)
    def pass_a(d):
        dbase = d * PITCH
        for c in range(BLK // L):
            plsc.store_scatter(pad1, [seg[c] + dbase],
                               slab[d, pl.ds(c * L, L)])

    @plsc.parallel_loop(0, BLK // 8, unroll=4)
    def pass_b(u):
        for vv in range(8):
            v = u * 8 + vv
            p = u * 4 + (vv >> 1)
            hb = (vv & 1) * DIM
            for c in range(DC):
                g = plsc.load_gather(pad1, [colbase[c] + v])
                rows_out[p, pl.ds(hb + L * c, L)] = g


def _tr_kernel(w_inT, w_outT, tail_in, tail_out, out_in, out_out,
               in0, in1, rows0, rows1, pad1,
               semr0, semr1, semw0, semw1):
    wid = lax.axis_index("s") * 2 + lax.axis_index("c")
    ins = (in0, in1)
    rows = (rows0, rows1)
    semr = (semr0, semr1)
    semw = (semw0, semw1)

    for src, tail, dst in ((w_inT, tail_in, out_in),
                           (w_outT, tail_out, out_out)):
        def rd(k, b):
            pltpu.async_copy(
                src.at[:, pl.ds((wid + NUM_WORKERS * k) * BLK, BLK)],
                ins[b], semr[b])

        def wr(k, b):
            pltpu.async_copy(
                rows[b],
                dst.at[pl.ds((wid + NUM_WORKERS * k) * (BLK // 2), BLK // 2),
                       :],
                semw[b])

        def wait_rd(k, b):
            pltpu.make_async_copy(
                src.at[:, pl.ds((wid + NUM_WORKERS * k) * BLK, BLK)],
                ins[b], semr[b]).wait()

        def wait_wr(k, b):
            pltpu.make_async_copy(
                rows[b],
                dst.at[pl.ds((wid + NUM_WORKERS * k) * (BLK // 2), BLK // 2),
                       :],
                semw[b]).wait()

        nk = jnp.where(wid < EXTRA, K_MAIN + 1, K_MAIN)
        rd(0, 0)
        rd(1, 1)

        def ring(g, carry):
            for b in range(2):
                k = 2 * g + b
                valid = k < nk

                @pl.when(valid)
                def _():
                    wait_rd(k, b)

                @pl.when(valid & (k >= 2))
                def _():
                    wait_wr(k - 2, b)

                @pl.when(valid)
                def _():
                    _transpose_block(ins[b], pad1, rows[b])
                    wr(k, b)

                @pl.when(valid & (k + 2 < nk))
                def _():
                    rd(k + 2, b)
            return carry

        lax.fori_loop(0, (K_MAIN + 2) // 2, ring, 0)
        for b in range(2):
            k_last = jnp.where((nk - 1) % 2 == b, nk - 1, nk - 2)
            wait_wr(k_last, b)

        @pl.when(wid == NUM_WORKERS - 1)
        def _():
            np_rows = TAIL_COLS // 2
            p0 = FULL_BLOCKS * (BLK // 2)
            pltpu.async_copy(tail, rows[1].at[pl.ds(0, np_rows), :], semr[1])
            pltpu.make_async_copy(tail, rows[1].at[pl.ds(0, np_rows), :],
                                  semr[1]).wait()
            pltpu.async_copy(rows[1].at[pl.ds(0, np_rows), :],
                             dst.at[pl.ds(p0, np_rows), :], semw[1])
            pltpu.make_async_copy(rows[1].at[pl.ds(0, np_rows), :],
                                  dst.at[pl.ds(p0, np_rows), :],
                                  semw[1]).wait()


_tr_call = functools.partial(
    pl.kernel,
    mesh=plsc.VectorSubcoreMesh(core_axis_name="c", subcore_axis_name="s"),
    compiler_params=pltpu.CompilerParams(needs_layout_passes=False),
    out_type=[jax.ShapeDtypeStruct((VOCAB // 2, PAIR), jnp.float32),
              jax.ShapeDtypeStruct((VOCAB // 2, PAIR), jnp.float32)],
    scratch_types=[
        pltpu.VMEM((DIM, BLK), jnp.float32),
        pltpu.VMEM((DIM, BLK), jnp.float32),
        pltpu.VMEM((BLK // 2, PAIR), jnp.float32),
        pltpu.VMEM((BLK // 2, PAIR), jnp.float32),
        pltpu.VMEM((DIM * PITCH,), jnp.float32),
        pltpu.SemaphoreType.DMA,
        pltpu.SemaphoreType.DMA,
        pltpu.SemaphoreType.DMA,
        pltpu.SemaphoreType.DMA,
    ],
)(_tr_kernel)


TPITCH = L + 1


def _tile_row(tile1, r, vec):
    plsc.store_scatter(tile1, [lax.iota(jnp.int32, L) + TPITCH * r], vec)


def _rowsum(tile1):
    rows17 = lax.iota(jnp.int32, L) * TPITCH
    acc = None
    for c in range(L):
        g = plsc.load_gather(tile1, [rows17 + c])
        acc = g if acc is None else acc + g
    return acc


GCHUNK = 32
GN_CHUNKS = PER_W // GCHUNK
GNEG = GCHUNK * K_NEG


def _sc_kernel(cen_idx, ctx_idx, neg_idx, w_in, w_out, pos_out, neg_out,
               cen_idx_v0, ctx_idx_v0, neg_idx_v0,
               cen_v0, ctx_v0, neg_v0, pos_st0, neg_st0,
               cen_idx_v1, ctx_idx_v1, neg_idx_v1,
               cen_v1, ctx_v1, neg_v1, pos_st1, neg_st1,
               tile, gsem0, gsem1, osem0, osem1):
    wid = lax.axis_index("s") * 2 + lax.axis_index("c")
    bufs = ((cen_idx_v0, ctx_idx_v0, neg_idx_v0, cen_v0, ctx_v0, neg_v0,
             pos_st0, neg_st0, gsem0, osem0),
            (cen_idx_v1, ctx_idx_v1, neg_idx_v1, cen_v1, ctx_v1, neg_v1,
             pos_st1, neg_st1, gsem1, osem1))

    def gather_cps(p):
        (civ, xiv, niv, cv, xv, nv, _, _, gsem, _) = bufs[p]
        cps = [pltpu.make_async_copy(w_in.at[civ], cv, gsem),
               pltpu.make_async_copy(w_out.at[xiv], xv, gsem)]
        for t in range(GNEG // IDX_SLICE):
            cps.append(pltpu.make_async_copy(
                w_out.at[niv.at[pl.ds(t * IDX_SLICE, IDX_SLICE)]],
                nv.at[pl.ds(t * IDX_SLICE, IDX_SLICE)], gsem))
        return cps

    def stage(ci, p):
        (civ, xiv, niv, *_rest) = bufs[p]
        base = wid * PER_W + ci * GCHUNK
        pltpu.sync_copy(cen_idx.at[pl.ds(base, GCHUNK)], civ)
        pltpu.sync_copy(ctx_idx.at[pl.ds(base, GCHUNK)], xiv)
        pltpu.sync_copy(neg_idx.at[pl.ds(base * K_NEG, GNEG)], niv)
        for cp in gather_cps(p):
            cp.start()

    def out_cps(ci, p):
        (*_rest, pst, nst, _, osem) = bufs[p]
        base = wid * PER_W + ci * GCHUNK
        return [pltpu.make_async_copy(pst, pos_out.at[pl.ds(base, GCHUNK)],
                                      osem),
                pltpu.make_async_copy(nst,
                                      neg_out.at[pl.ds(base * K_NEG, GNEG)],
                                      osem)]

    def compute(ci, p):
        (_, _, _, cen_v, ctx_v, neg_v, pos_st, neg_st, _, _) = bufs[p]

        def pos_group(i, c2):
            for j in range(L):
                e = i * L + j
                acc = None
                for c in range(DC):
                    a = cen_v[e, pl.ds(L * c, L)]
                    b = ctx_v[e, pl.ds(L * c, L)]
                    acc = a * b if acc is None else acc + a * b
                _tile_row(tile, j, acc)
            pos_st[pl.ds(i * L, L)] = _rowsum(tile)
            return c2

        lax.fori_loop(0, GCHUNK // L, pos_group, 0)

        def neg_group(g, c2):
            for eo in range(4):
                e = g * 4 + eo
                cen_c = [cen_v[e, pl.ds(L * c, L)] for c in range(DC)]
                for k in range(K_NEG):
                    sl = eo * K_NEG + k
                    srow = g * 80 + sl
                    acc = None
                    for c in range(DC):
                        n = neg_v[srow, pl.ds(L * c, L)]
                        acc = (n * cen_c[c] if acc is None
                               else acc + n * cen_c[c])
                    _tile_row(tile, sl % L, acc)
                    if sl % L == L - 1:
                        neg_st[pl.ds(g * 80 + (sl // L) * L, L)] = \
                            _rowsum(tile)
            return c2

        lax.fori_loop(0, GNEG // 80, neg_group, 0)

        for cp in out_cps(ci, p):
            cp.start()

    stage(0, 0)
    stage(1, 1)

    def pipe(gi, carry):
        for p in range(2):
            ci = 2 * gi + p
            for cp in gather_cps(p):
                cp.wait()

            @pl.when(gi > 0)
            def _():
                for cp in out_cps(ci - 2, p):
                    cp.wait()

            compute(ci, p)

            @pl.when(ci + 2 < GN_CHUNKS)
            def _():
                stage(ci + 2, p)
        return carry

    lax.fori_loop(0, GN_CHUNKS // 2, pipe, 0)
    for cp in out_cps(GN_CHUNKS - 2, 0) + out_cps(GN_CHUNKS - 1, 1):
        cp.wait()


_GBUFS = [
    pltpu.VMEM((GCHUNK,), jnp.int32),
    pltpu.VMEM((GCHUNK,), jnp.int32),
    pltpu.VMEM((GNEG,), jnp.int32),
    pltpu.VMEM((GCHUNK, DIM), jnp.float32),
    pltpu.VMEM((GCHUNK, DIM), jnp.float32),
    pltpu.VMEM((GNEG, DIM), jnp.float32),
    pltpu.VMEM((GCHUNK,), jnp.float32),
    pltpu.VMEM((GNEG,), jnp.float32),
]

_sc_call = functools.partial(
    pl.kernel,
    mesh=plsc.VectorSubcoreMesh(core_axis_name="c", subcore_axis_name="s"),
    compiler_params=pltpu.CompilerParams(needs_layout_passes=False,
                                         use_tc_tiling_on_sc=False),
    out_type=[jax.ShapeDtypeStruct((B,), jnp.float32),
              jax.ShapeDtypeStruct((B * K_NEG,), jnp.float32)],
    scratch_types=(
        _GBUFS + _GBUFS + [
            pltpu.VMEM((L * TPITCH,), jnp.float32),
            pltpu.SemaphoreType.DMA,
            pltpu.SemaphoreType.DMA,
            pltpu.SemaphoreType.DMA,
            pltpu.SemaphoreType.DMA,
        ]
    ),
)(_sc_kernel)


@jax.jit
def kernel(center_words, context_words, negative_samples, W_in, W_out):
    cen = center_words.reshape(B).astype(jnp.int32)
    ctx = context_words.reshape(B).astype(jnp.int32)
    neg = negative_samples.reshape(B * K_NEG).astype(jnp.int32)
    t0 = FULL_BLOCKS * BLK
    tin = W_in[t0:, :].reshape(TAIL_COLS // 2, PAIR)
    tout = W_out[t0:, :].reshape(TAIL_COLS // 2, PAIR)
    w_in2, w_out2 = _tr_call(W_in.T, W_out.T, tin, tout)
    pos, negf = _sc_call(cen, ctx, neg,
                         w_in2.reshape(VOCAB, DIM),
                         w_out2.reshape(VOCAB, DIM))
    return pos, negf.reshape(B, K_NEG)
